# baseline (speedup 1.0000x reference)
import functools

import jax
import jax.numpy as jnp
from jax import lax
from jax.experimental import pallas as pl
from jax.experimental.pallas import tpu as pltpu
from jax.experimental.pallas import tpu_sc as plsc

LEAK = 0.2
EPS = 1e-5
K = 16
NWORKERS = 32
PTS_PER_CHUNK = 8


def _proj_body(x_ref, wc_ref, y_ref, z_ref, *, c_out):
    xb = x_ref[...]
    yz = lax.dot_general(xb, wc_ref[...], (((1,), (0,)), ((), ())),
                         preferred_element_type=jnp.float32,
                         precision=lax.Precision.HIGHEST)
    y_ref[...] = yz[:, :c_out]
    z_ref[...] = yz[:, c_out:]


def _knn_body(xb_ref, xs_ref, xst_ref, idx_ref, *, k, s_seg):
    seg = pl.program_id(0)
    xb = xb_ref[...]
    xs = xs_ref[...]
    xst = xst_ref[...]
    dots = lax.dot_general(xb, xs, (((1,), (1,)), ((), ())),
                           preferred_element_type=jnp.float32)
    sq_i = jnp.sum(xb * xb, axis=1, keepdims=True)
    sq_j = jnp.sum(xst * xst, axis=0, keepdims=True)
    d2 = (sq_i + sq_j) - 2.0 * dots
    iota = lax.broadcasted_iota(jnp.int32, d2.shape, 1)
    big = jnp.float32(3.0e38)
    idx_cols = []
    for _ in range(k):
        mv = jnp.min(d2, axis=1, keepdims=True)
        cand = jnp.where(d2 == mv, iota, s_seg)
        am = jnp.min(cand, axis=1, keepdims=True)
        d2 = jnp.where(iota == am, big, d2)
        idx_cols.append(am)
    idx_ref[...] = jnp.concatenate(idx_cols, axis=1) + seg * s_seg


def _sc_body(y_hbm, idx_hbm, out_hbm, idx_v, gbuf, stage,
             gs0, gs1, os0, os1, *, c_out, chunks_per_w, k, pts):
    gsem = (gs0, gs1)
    osem = (os0, os1)
    wid = lax.axis_index("s") * 2 + lax.axis_index("c")
    rows_per_chunk = pts * k
    chunk0 = wid * chunks_per_w

    pltpu.sync_copy(idx_hbm.at[pl.ds(chunk0, chunks_per_w)], idx_v)

    def fire_gather(c, b):
        pltpu.async_copy(y_hbm.at[idx_v.at[c]], gbuf.at[b], gsem[b])

    def wait_gather(c, b):
        pltpu.make_async_copy(y_hbm.at[idx_v.at[c]], gbuf.at[b], gsem[b]).wait()

    def fire_out(c, b):
        base = (chunk0 + c) * pts
        pltpu.async_copy(stage.at[b], out_hbm.at[pl.ds(base, pts)], osem[b])

    def wait_out(c, b):
        base = (chunk0 + c) * pts
        pltpu.make_async_copy(stage.at[b], out_hbm.at[pl.ds(base, pts)],
                              osem[b]).wait()

    fire_gather(0, 0)
    fire_gather(1, 1)

    def do_phase(c, b):
        wait_gather(c, b)

        @pl.when(c >= 2)
        def _():
            wait_out(c - 2, b)

        gb = gbuf.at[b]
        st = stage.at[b]

        @pl.loop(0, pts)
        def _pt(p):
            r0 = p * k
            for v in range(c_out // 16):
                sl = pl.ds(v * 16, 16)
                a0 = gb[r0, sl]
                amax = a0
                amin = a0
                asum = a0
                asq = a0 * a0
                for r in range(1, k):
                    av = gb[r0 + r, sl]
                    amax = jnp.maximum(amax, av)
                    amin = jnp.minimum(amin, av)
                    asum = asum + av
                    asq = asq + av * av
                st[p, pl.ds(v * 16, 16)] = amax
                st[p, pl.ds(c_out + v * 16, 16)] = amin
                st[p, pl.ds(2 * c_out + v * 16, 16)] = asum
                st[p, pl.ds(3 * c_out + v * 16, 16)] = asq

        fire_out(c, b)

        @pl.when(c + 2 < chunks_per_w)
        def _():
            fire_gather(c + 2, b)

    @pl.loop(0, chunks_per_w, step=2)
    def _chunk(c):
        do_phase(c, 0)
        do_phase(c + 1, 1)

    wait_out(chunks_per_w - 2, 0)
    wait_out(chunks_per_w - 1, 1)


def _stats_body(sc_ref, z_ref, gam_ref, bet_ref, out_ref, acc_ref,
                *, nblocks, n_total, k, c_out):
    i = pl.program_id(0)

    @pl.when(i == 0)
    def _init():
        acc_ref[...] = jnp.zeros_like(acc_ref)

    blk = sc_ref[...]
    g = blk[:, 2 * c_out:3 * c_out]
    q = blk[:, 3 * c_out:4 * c_out]
    z = z_ref[...]
    acc_ref[0:1, :] += jnp.sum(g, axis=0, keepdims=True)
    acc_ref[1:2, :] += jnp.sum(q, axis=0, keepdims=True)
    acc_ref[2:3, :] += jnp.sum(z * g, axis=0, keepdims=True)
    acc_ref[3:4, :] += jnp.sum(z, axis=0, keepdims=True)
    acc_ref[4:5, :] += jnp.sum(z * z, axis=0, keepdims=True)

    @pl.when(i == nblocks - 1)
    def _fin():
        cnt = jnp.float32(n_total * k)
        kf = jnp.float32(k)
        sum_g = acc_ref[0:1, :]
        sum_q = acc_ref[1:2, :]
        sum_zg = acc_ref[2:3, :]
        sum_z = acc_ref[3:4, :]
        sum_z2 = acc_ref[4:5, :]
        mean = (sum_g + kf * sum_z) / cnt
        e2 = (sum_q + 2.0 * sum_zg + kf * sum_z2) / cnt
        var = e2 - mean * mean
        scale = gam_ref[...] * lax.rsqrt(var + EPS)
        shift = bet_ref[...] - mean * scale
        out_ref[0:1, :] = scale
        out_ref[1:2, :] = shift


def _fin_body(z_ref, sc_ref, stat_ref, out_ref, *, c_out):
    scale = stat_ref[0:1, :]
    shift = stat_ref[1:2, :]
    blk = sc_ref[...]
    m = jnp.where(scale >= 0.0, blk[:, :c_out], blk[:, c_out:2 * c_out])
    v = (z_ref[...] + m) * scale + shift
    out_ref[...] = jnp.where(v >= 0.0, v, LEAK * v)


def kernel(p, x, o, W, gamma, beta):
    n, d = x.shape
    bseg = o.shape[0]
    s_seg = n // bseg
    c_out = W.shape[1]
    k = K

    w1 = W[:d]
    w2 = W[d:]
    wc = jnp.concatenate([w1, w2 - w1], axis=1)
    xt = x.T

    rb = 512
    f32 = jnp.float32
    y, z = pl.pallas_call(
        functools.partial(_proj_body, c_out=c_out),
        grid=(n // rb,),
        in_specs=[
            pl.BlockSpec((rb, d), lambda i: (i, 0)),
            pl.BlockSpec((d, 2 * c_out), lambda i: (0, 0)),
        ],
        out_specs=[
            pl.BlockSpec((rb, c_out), lambda i: (i, 0)),
            pl.BlockSpec((rb, c_out), lambda i: (i, 0)),
        ],
        out_shape=[
            jax.ShapeDtypeStruct((n, c_out), f32),
            jax.ShapeDtypeStruct((n, c_out), f32),
        ],
    )(x, wc)

    r = 256
    nrb = s_seg // r
    idx = pl.pallas_call(
        functools.partial(_knn_body, k=k, s_seg=s_seg),
        grid=(bseg, nrb),
        in_specs=[
            pl.BlockSpec((r, d), lambda s, rr: (s * nrb + rr, 0)),
            pl.BlockSpec((s_seg, d), lambda s, rr: (s, 0)),
            pl.BlockSpec((d, s_seg), lambda s, rr: (0, s)),
        ],
        out_specs=pl.BlockSpec((r, k), lambda s, rr: (s * nrb + rr, 0)),
        out_shape=jax.ShapeDtypeStruct((n, k), jnp.int32),
    )(x, x, xt)

    pts = PTS_PER_CHUNK
    rows_per_chunk = pts * k
    n_chunks = n // pts
    chunks_per_w = n_chunks // NWORKERS
    idx2d = idx.reshape(n_chunks, rows_per_chunk)

    mesh = plsc.VectorSubcoreMesh(core_axis_name="c", subcore_axis_name="s")
    sc_out = pl.kernel(
        functools.partial(_sc_body, c_out=c_out, chunks_per_w=chunks_per_w,
                          k=k, pts=pts),
        out_type=jax.ShapeDtypeStruct((n, 4 * c_out), f32),
        mesh=mesh,
        scratch_types=[
            pltpu.VMEM((chunks_per_w, rows_per_chunk), jnp.int32),
            pltpu.VMEM((2, rows_per_chunk, c_out), f32),
            pltpu.VMEM((2, pts, 4 * c_out), f32),
            pltpu.SemaphoreType.DMA,
            pltpu.SemaphoreType.DMA,
            pltpu.SemaphoreType.DMA,
            pltpu.SemaphoreType.DMA,
        ],
        compiler_params=pltpu.CompilerParams(use_tc_tiling_on_sc=False),
    )(y, idx2d)

    nblocks = n // rb
    stat = pl.pallas_call(
        functools.partial(_stats_body, nblocks=nblocks, n_total=n, k=k,
                          c_out=c_out),
        grid=(nblocks,),
        in_specs=[
            pl.BlockSpec((rb, 4 * c_out), lambda i: (i, 0)),
            pl.BlockSpec((rb, c_out), lambda i: (i, 0)),
            pl.BlockSpec((1, c_out), lambda i: (0, 0)),
            pl.BlockSpec((1, c_out), lambda i: (0, 0)),
        ],
        out_specs=pl.BlockSpec((2, c_out), lambda i: (0, 0)),
        out_shape=jax.ShapeDtypeStruct((2, c_out), f32),
        scratch_shapes=[pltpu.VMEM((8, c_out), f32)],
    )(sc_out, z, gamma.reshape(1, c_out), beta.reshape(1, c_out))

    out = pl.pallas_call(
        functools.partial(_fin_body, c_out=c_out),
        grid=(nblocks,),
        in_specs=[
            pl.BlockSpec((rb, c_out), lambda i: (i, 0)),
            pl.BlockSpec((rb, 4 * c_out), lambda i: (i, 0)),
            pl.BlockSpec((2, c_out), lambda i: (0, 0)),
        ],
        out_specs=pl.BlockSpec((rb, c_out), lambda i: (i, 0)),
        out_shape=jax.ShapeDtypeStruct((n, c_out), f32),
    )(z, sc_out, stat)
    return out


# speedup vs baseline: 1.0318x; 1.0318x over previous
import functools

import jax
import jax.numpy as jnp
from jax import lax
from jax.experimental import pallas as pl
from jax.experimental.pallas import tpu as pltpu
from jax.experimental.pallas import tpu_sc as plsc

LEAK = 0.2
EPS = 1e-5
K = 16
NWORKERS = 32
PTS_PER_CHUNK = 8


def _proj_body(x_ref, wc_ref, y_ref, z_ref, *, c_out):
    xb = x_ref[...]
    yz = lax.dot_general(xb, wc_ref[...], (((1,), (0,)), ((), ())),
                         preferred_element_type=jnp.float32,
                         precision=lax.Precision.HIGHEST)
    y_ref[...] = yz[:, :c_out]
    z_ref[...] = yz[:, c_out:]


def _knn_body(xb_ref, xs_ref, xst_ref, idx_ref, *, k, s_seg):
    seg = pl.program_id(0)
    xb = xb_ref[...]
    xs = xs_ref[...]
    xst = xst_ref[...]
    dots = lax.dot_general(xb, xs, (((1,), (1,)), ((), ())),
                           preferred_element_type=jnp.float32)
    sq_i = jnp.sum(xb * xb, axis=1, keepdims=True)
    sq_j = jnp.sum(xst * xst, axis=0, keepdims=True)
    d2 = (sq_i + sq_j) - 2.0 * dots
    big = jnp.float32(3.0e38)
    nch = d2.shape[1] // 128
    iota = lax.broadcasted_iota(jnp.int32, (d2.shape[0], 128), 1)
    ds = [d2[:, c * 128:(c + 1) * 128] for c in range(nch)]
    io = [iota + c * 128 for c in range(nch)]
    idx_cols = []
    for _ in range(k):
        v = ds[0]
        ii = io[0]
        for c in range(1, nch):
            take = ds[c] < v
            v = jnp.where(take, ds[c], v)
            ii = jnp.where(take, io[c], ii)
        mv = jnp.min(v, axis=1, keepdims=True)
        cand = jnp.where(v == mv, ii, s_seg)
        am = jnp.min(cand, axis=1, keepdims=True)
        for c in range(nch):
            ds[c] = jnp.where(io[c] == am, big, ds[c])
        idx_cols.append(am)
    idx_ref[...] = jnp.concatenate(idx_cols, axis=1) + seg * s_seg


def _sc_body(y_hbm, idx_hbm, out_hbm, idx_v, gbuf, stage,
             gs0, gs1, os0, os1, *, c_out, chunks_per_w, k, pts):
    gsem = (gs0, gs1)
    osem = (os0, os1)
    wid = lax.axis_index("s") * 2 + lax.axis_index("c")
    rows_per_chunk = pts * k
    chunk0 = wid * chunks_per_w

    pltpu.sync_copy(idx_hbm.at[pl.ds(chunk0, chunks_per_w)], idx_v)

    def fire_gather(c, b):
        pltpu.async_copy(y_hbm.at[idx_v.at[c]], gbuf.at[b], gsem[b])

    def wait_gather(c, b):
        pltpu.make_async_copy(y_hbm.at[idx_v.at[c]], gbuf.at[b], gsem[b]).wait()

    def fire_out(c, b):
        base = (chunk0 + c) * pts
        pltpu.async_copy(stage.at[b], out_hbm.at[pl.ds(base, pts)], osem[b])

    def wait_out(c, b):
        base = (chunk0 + c) * pts
        pltpu.make_async_copy(stage.at[b], out_hbm.at[pl.ds(base, pts)],
                              osem[b]).wait()

    fire_gather(0, 0)
    fire_gather(1, 1)

    def do_phase(c, b):
        wait_gather(c, b)

        @pl.when(c >= 2)
        def _():
            wait_out(c - 2, b)

        gb = gbuf.at[b]
        st = stage.at[b]

        @pl.loop(0, pts)
        def _pt(p):
            r0 = p * k
            for v in range(c_out // 16):
                sl = pl.ds(v * 16, 16)
                a0 = gb[r0, sl]
                amax = a0
                amin = a0
                asum = a0
                asq = a0 * a0
                for r in range(1, k):
                    av = gb[r0 + r, sl]
                    amax = jnp.maximum(amax, av)
                    amin = jnp.minimum(amin, av)
                    asum = asum + av
                    asq = asq + av * av
                st[p, pl.ds(v * 16, 16)] = amax
                st[p, pl.ds(c_out + v * 16, 16)] = amin
                st[p, pl.ds(2 * c_out + v * 16, 16)] = asum
                st[p, pl.ds(3 * c_out + v * 16, 16)] = asq

        fire_out(c, b)

        @pl.when(c + 2 < chunks_per_w)
        def _():
            fire_gather(c + 2, b)

    @pl.loop(0, chunks_per_w, step=2)
    def _chunk(c):
        do_phase(c, 0)
        do_phase(c + 1, 1)

    wait_out(chunks_per_w - 2, 0)
    wait_out(chunks_per_w - 1, 1)


def _stats_body(sc_ref, z_ref, gam_ref, bet_ref, out_ref, acc_ref,
                *, nblocks, n_total, k, c_out):
    i = pl.program_id(0)

    @pl.when(i == 0)
    def _init():
        acc_ref[...] = jnp.zeros_like(acc_ref)

    blk = sc_ref[...]
    g = blk[:, 2 * c_out:3 * c_out]
    q = blk[:, 3 * c_out:4 * c_out]
    z = z_ref[...]
    acc_ref[0:1, :] += jnp.sum(g, axis=0, keepdims=True)
    acc_ref[1:2, :] += jnp.sum(q, axis=0, keepdims=True)
    acc_ref[2:3, :] += jnp.sum(z * g, axis=0, keepdims=True)
    acc_ref[3:4, :] += jnp.sum(z, axis=0, keepdims=True)
    acc_ref[4:5, :] += jnp.sum(z * z, axis=0, keepdims=True)

    @pl.when(i == nblocks - 1)
    def _fin():
        cnt = jnp.float32(n_total * k)
        kf = jnp.float32(k)
        sum_g = acc_ref[0:1, :]
        sum_q = acc_ref[1:2, :]
        sum_zg = acc_ref[2:3, :]
        sum_z = acc_ref[3:4, :]
        sum_z2 = acc_ref[4:5, :]
        mean = (sum_g + kf * sum_z) / cnt
        e2 = (sum_q + 2.0 * sum_zg + kf * sum_z2) / cnt
        var = e2 - mean * mean
        scale = gam_ref[...] * lax.rsqrt(var + EPS)
        shift = bet_ref[...] - mean * scale
        out_ref[0:1, :] = scale
        out_ref[1:2, :] = shift


def _fin_body(z_ref, sc_ref, stat_ref, out_ref, *, c_out):
    scale = stat_ref[0:1, :]
    shift = stat_ref[1:2, :]
    blk = sc_ref[...]
    m = jnp.where(scale >= 0.0, blk[:, :c_out], blk[:, c_out:2 * c_out])
    v = (z_ref[...] + m) * scale + shift
    out_ref[...] = jnp.where(v >= 0.0, v, LEAK * v)


def kernel(p, x, o, W, gamma, beta):
    n, d = x.shape
    bseg = o.shape[0]
    s_seg = n // bseg
    c_out = W.shape[1]
    k = K

    w1 = W[:d]
    w2 = W[d:]
    wc = jnp.concatenate([w1, w2 - w1], axis=1)
    xt = x.T

    rb = 512
    f32 = jnp.float32
    y, z = pl.pallas_call(
        functools.partial(_proj_body, c_out=c_out),
        grid=(n // rb,),
        in_specs=[
            pl.BlockSpec((rb, d), lambda i: (i, 0)),
            pl.BlockSpec((d, 2 * c_out), lambda i: (0, 0)),
        ],
        out_specs=[
            pl.BlockSpec((rb, c_out), lambda i: (i, 0)),
            pl.BlockSpec((rb, c_out), lambda i: (i, 0)),
        ],
        out_shape=[
            jax.ShapeDtypeStruct((n, c_out), f32),
            jax.ShapeDtypeStruct((n, c_out), f32),
        ],
    )(x, wc)

    r = 256
    nrb = s_seg // r
    idx = pl.pallas_call(
        functools.partial(_knn_body, k=k, s_seg=s_seg),
        grid=(bseg, nrb),
        in_specs=[
            pl.BlockSpec((r, d), lambda s, rr: (s * nrb + rr, 0)),
            pl.BlockSpec((s_seg, d), lambda s, rr: (s, 0)),
            pl.BlockSpec((d, s_seg), lambda s, rr: (0, s)),
        ],
        out_specs=pl.BlockSpec((r, k), lambda s, rr: (s * nrb + rr, 0)),
        out_shape=jax.ShapeDtypeStruct((n, k), jnp.int32),
    )(x, x, xt)

    pts = PTS_PER_CHUNK
    rows_per_chunk = pts * k
    n_chunks = n // pts
    chunks_per_w = n_chunks // NWORKERS
    idx2d = idx.reshape(n_chunks, rows_per_chunk)

    mesh = plsc.VectorSubcoreMesh(core_axis_name="c", subcore_axis_name="s")
    sc_out = pl.kernel(
        functools.partial(_sc_body, c_out=c_out, chunks_per_w=chunks_per_w,
                          k=k, pts=pts),
        out_type=jax.ShapeDtypeStruct((n, 4 * c_out), f32),
        mesh=mesh,
        scratch_types=[
            pltpu.VMEM((chunks_per_w, rows_per_chunk), jnp.int32),
            pltpu.VMEM((2, rows_per_chunk, c_out), f32),
            pltpu.VMEM((2, pts, 4 * c_out), f32),
            pltpu.SemaphoreType.DMA,
            pltpu.SemaphoreType.DMA,
            pltpu.SemaphoreType.DMA,
            pltpu.SemaphoreType.DMA,
        ],
        compiler_params=pltpu.CompilerParams(use_tc_tiling_on_sc=False),
    )(y, idx2d)

    nblocks = n // rb
    stat = pl.pallas_call(
        functools.partial(_stats_body, nblocks=nblocks, n_total=n, k=k,
                          c_out=c_out),
        grid=(nblocks,),
        in_specs=[
            pl.BlockSpec((rb, 4 * c_out), lambda i: (i, 0)),
            pl.BlockSpec((rb, c_out), lambda i: (i, 0)),
            pl.BlockSpec((1, c_out), lambda i: (0, 0)),
            pl.BlockSpec((1, c_out), lambda i: (0, 0)),
        ],
        out_specs=pl.BlockSpec((2, c_out), lambda i: (0, 0)),
        out_shape=jax.ShapeDtypeStruct((2, c_out), f32),
        scratch_shapes=[pltpu.VMEM((8, c_out), f32)],
    )(sc_out, z, gamma.reshape(1, c_out), beta.reshape(1, c_out))

    out = pl.pallas_call(
        functools.partial(_fin_body, c_out=c_out),
        grid=(nblocks,),
        in_specs=[
            pl.BlockSpec((rb, c_out), lambda i: (i, 0)),
            pl.BlockSpec((rb, 4 * c_out), lambda i: (i, 0)),
            pl.BlockSpec((2, c_out), lambda i: (0, 0)),
        ],
        out_specs=pl.BlockSpec((rb, c_out), lambda i: (i, 0)),
        out_shape=jax.ShapeDtypeStruct((n, c_out), f32),
    )(z, sc_out, stat)
    return out
